# baseline (speedup 1.0000x reference)
import jax
import jax.numpy as jnp
from jax.experimental import pallas as pl


def kernel(membership, table):
    B, S, D = membership.shape
    E = table.shape[1]
    BB = 8

    def body(m_ref, t_ref, out_ref):
        from jax import lax
        m = lax.broadcast_in_dim(m_ref[...], (BB, S, D, E), (0, 1, 2))
        out_ref[...] = jnp.where(m == 1, t_ref[1], t_ref[0])

    return pl.pallas_call(
        body,
        grid=(B // BB,),
        in_specs=[
            pl.BlockSpec((BB, S, D), lambda i: (i, 0, 0)),
            pl.BlockSpec((2, E), lambda i: (0, 0)),
        ],
        out_specs=pl.BlockSpec((BB, S, D, E), lambda i: (i, 0, 0, 0)),
        out_shape=jax.ShapeDtypeStruct((B, S, D, E), jnp.float32),
    )(membership.astype(jnp.int32), table)
